# baseline (speedup 1.0000x reference)
import functools

import jax
import jax.numpy as jnp
from jax import lax
from jax.experimental import pallas as pl
from jax.experimental.pallas import tpu as pltpu
from jax.experimental.pallas import tpu_sc as plsc

_INDICES = (1,) * 10
_N_OUT = len(_INDICES)
_LANES = 16
_ROWS, _COLS = 100000, 64
_T_ROW = _INDICES[0] % _COLS
_T_COL = _INDICES[0] // _COLS
assert all(i % _COLS == _T_ROW and i // _COLS == _T_COL for i in _INDICES)

_MESH = plsc.VectorSubcoreMesh(
    core_axis_name="c", subcore_axis_name="s", num_cores=1, num_subcores=1
)


@functools.partial(
    pl.kernel,
    out_type=jax.ShapeDtypeStruct((_N_OUT,), jnp.float32),
    mesh=_MESH,
    scratch_types=[
        pltpu.VMEM((_LANES,), jnp.float32),
        pltpu.VMEM((_LANES,), jnp.float32),
    ],
    compiler_params=pltpu.CompilerParams(use_tc_tiling_on_sc=True),
)
def _gather_sc(xt_hbm, out_hbm, row_v, val_v):
    c = lax.axis_index("c")
    s = lax.axis_index("s")

    is_w0 = jnp.logical_and(c == 0, s == 0)

    @pl.when(is_w0)
    def _():
        pltpu.sync_copy(xt_hbm.at[_T_ROW, pl.ds(_T_COL, _LANES)], row_v)

    idx = jnp.minimum(lax.iota(jnp.int32, _LANES), 0)
    window = row_v[...]
    val_v[...] = lax.gather(
        window,
        idx[:, None],
        lax.GatherDimensionNumbers(
            offset_dims=(),
            collapsed_slice_dims=(0,),
            start_index_map=(0,),
        ),
        slice_sizes=(1,),
        mode=lax.GatherScatterMode.PROMISE_IN_BOUNDS,
    )

    @pl.when(is_w0)
    def _():
        pltpu.sync_copy(val_v.at[pl.ds(0, _N_OUT)], out_hbm)


def kernel(x):
    return _gather_sc(x.T)


# speedup vs baseline: 1.0017x; 1.0017x over previous
import functools

import jax
import jax.numpy as jnp
from jax import lax
from jax.experimental import pallas as pl
from jax.experimental.pallas import tpu as pltpu
from jax.experimental.pallas import tpu_sc as plsc

_INDICES = (1,) * 10
_N_OUT = len(_INDICES)
_LANES = 16
_ROWS, _COLS = 100000, 64
_T_ROW = _INDICES[0] % _COLS
_T_COL = _INDICES[0] // _COLS
assert all(i % _COLS == _T_ROW and i // _COLS == _T_COL for i in _INDICES)

_MESH = plsc.VectorSubcoreMesh(
    core_axis_name="c", subcore_axis_name="s", num_cores=1, num_subcores=1
)


@functools.partial(
    pl.kernel,
    out_type=jax.ShapeDtypeStruct((_N_OUT,), jnp.float32),
    mesh=_MESH,
    scratch_types=[
        pltpu.VMEM((_LANES,), jnp.float32),
        pltpu.VMEM((_LANES,), jnp.float32),
    ],
    compiler_params=pltpu.CompilerParams(
        use_tc_tiling_on_sc=True, skip_device_barrier=True
    ),
)
def _gather_sc(xt_hbm, out_hbm, row_v, val_v):
    c = lax.axis_index("c")
    s = lax.axis_index("s")

    is_w0 = jnp.logical_and(c == 0, s == 0)

    @pl.when(is_w0)
    def _():
        pltpu.sync_copy(xt_hbm.at[_T_ROW, pl.ds(_T_COL, _LANES)], row_v)

    idx = jnp.minimum(lax.iota(jnp.int32, _LANES), 0)
    window = row_v[...]
    val_v[...] = lax.gather(
        window,
        idx[:, None],
        lax.GatherDimensionNumbers(
            offset_dims=(),
            collapsed_slice_dims=(0,),
            start_index_map=(0,),
        ),
        slice_sizes=(1,),
        mode=lax.GatherScatterMode.PROMISE_IN_BOUNDS,
    )

    @pl.when(is_w0)
    def _():
        pltpu.sync_copy(val_v.at[pl.ds(0, _N_OUT)], out_hbm)


def kernel(x):
    return _gather_sc(x.T)


# speedup vs baseline: 1.0028x; 1.0011x over previous
import functools

import jax
import jax.numpy as jnp
from jax import lax
from jax.experimental import pallas as pl
from jax.experimental.pallas import tpu as pltpu
from jax.experimental.pallas import tpu_sc as plsc

_INDICES = (1,) * 10
_N_OUT = len(_INDICES)
_LANES = 16
_ROWS, _COLS = 100000, 64
_T_ROW = _INDICES[0] % _COLS
_T_COL = _INDICES[0] // _COLS
assert all(i % _COLS == _T_ROW and i // _COLS == _T_COL for i in _INDICES)

_MESH = plsc.VectorSubcoreMesh(
    core_axis_name="c", subcore_axis_name="s", num_cores=1, num_subcores=1
)


@functools.partial(
    pl.kernel,
    out_type=jax.ShapeDtypeStruct((_N_OUT,), jnp.float32),
    mesh=_MESH,
    scratch_types=[
        pltpu.VMEM((_LANES,), jnp.float32),
        pltpu.VMEM((_LANES,), jnp.float32),
    ],
    compiler_params=pltpu.CompilerParams(use_tc_tiling_on_sc=True),
)
def _gather_sc(xt_hbm, out_hbm, row_v, val_v):
    pltpu.sync_copy(xt_hbm.at[_T_ROW, pl.ds(_T_COL, _LANES)], row_v)

    idx = jnp.minimum(lax.iota(jnp.int32, _LANES), 0)
    val_v[...] = lax.gather(
        row_v[...],
        idx[:, None],
        lax.GatherDimensionNumbers(
            offset_dims=(),
            collapsed_slice_dims=(0,),
            start_index_map=(0,),
        ),
        slice_sizes=(1,),
        mode=lax.GatherScatterMode.PROMISE_IN_BOUNDS,
    )

    pltpu.sync_copy(val_v.at[pl.ds(0, _N_OUT)], out_hbm)


def kernel(x):
    return _gather_sc(x.T)
